# baseline (speedup 1.0000x reference)
import functools

import jax
import jax.numpy as jnp
from jax import lax
from jax.experimental import pallas as pl
from jax.experimental.pallas import tpu as pltpu
from jax.experimental.pallas import tpu_sc as plsc

D = 768
KV = D // 16
B = 128
L = 512
NC = 2
NS = 16
NW = NC * NS
POS_W = L // NW
TOK_W = B * POS_W
SEQ_CHUNK = 4
CROWS = SEQ_CHUNK * POS_W
NCHUNK = B // SEQ_CHUNK
EPS = 1e-12


def _body(txt_ref, we_ref, pe_ref, te_ref, g_ref, b_ref, out_ref,
          idx_v, pe_v, te_v, g_v, b_v, rows_v, sem):
    c = lax.axis_index("c")
    s = lax.axis_index("s")
    wid = s * NC + c

    pltpu.sync_copy(txt_ref.at[wid], idx_v)
    pltpu.sync_copy(pe_ref.at[pl.ds(wid * POS_W, POS_W)], pe_v)
    pltpu.sync_copy(te_ref.at[pl.ds(0, 1)], te_v)
    pltpu.sync_copy(g_ref, g_v)
    pltpu.sync_copy(b_ref, b_v)

    def fold(r, carry):
        for k in range(KV):
            sl = pl.ds(k * 16, 16)
            pe_v[r, sl] = pe_v[r, sl] + te_v[0, sl]
        return carry
    lax.fori_loop(0, POS_W, fold, 0)

    inv_d = jnp.float32(1.0 / D)

    def row_ln(r, carry):
        p = lax.rem(r, POS_W)
        acc = jnp.zeros((16,), jnp.float32)
        acc2 = jnp.zeros((16,), jnp.float32)
        for k in range(KV):
            sl = pl.ds(k * 16, 16)
            x = rows_v[r, sl] + pe_v[p, sl]
            rows_v[r, sl] = x
            acc = acc + x
            acc2 = acc2 + x * x
        s1 = jnp.sum(acc)
        s2 = jnp.sum(acc2)
        mu = s1 * inv_d
        var = s2 * inv_d - mu * mu + jnp.float32(EPS)
        v = jnp.full((16,), var, jnp.float32)
        i = plsc.bitcast(v, jnp.int32)
        y = plsc.bitcast(jnp.int32(0x5F3759DF) - (i >> 1), jnp.float32)
        half_v = v * jnp.float32(0.5)
        for _ in range(3):
            y = y * (jnp.float32(1.5) - half_v * y * y)
        muv = jnp.full((16,), mu, jnp.float32)
        for k in range(KV):
            sl = pl.ds(k * 16, 16)
            x = rows_v[r, sl]
            rows_v[r, sl] = (x - muv) * y * g_v[sl] + b_v[sl]
        return carry

    def chunk(ch, carry):
        pltpu.async_copy(
            we_ref.at[idx_v.at[pl.ds(ch * CROWS, CROWS)]], rows_v, sem
        ).wait()
        lax.fori_loop(0, CROWS, row_ln, 0)
        for j in range(SEQ_CHUNK):
            seq = ch * SEQ_CHUNK + j
            pltpu.sync_copy(
                rows_v.at[pl.ds(j * POS_W, POS_W)],
                out_ref.at[pl.ds(seq * L + wid * POS_W, POS_W)],
            )
        return carry

    lax.fori_loop(0, NCHUNK, chunk, 0)


@jax.jit
def _run(txt_w, word_embeddings, position_embeddings, token_type_embeddings,
         ln_gamma, ln_beta):
    mesh = plsc.VectorSubcoreMesh(core_axis_name="c", subcore_axis_name="s")
    k = pl.kernel(
        _body,
        out_type=jax.ShapeDtypeStruct((B * L, D), jnp.float32),
        mesh=mesh,
        compiler_params=pltpu.CompilerParams(needs_layout_passes=False),
        scratch_types=[
            pltpu.VMEM((TOK_W,), jnp.int32),
            pltpu.VMEM((POS_W, D), jnp.float32),
            pltpu.VMEM((1, D), jnp.float32),
            pltpu.VMEM((D,), jnp.float32),
            pltpu.VMEM((D,), jnp.float32),
            pltpu.VMEM((CROWS, D), jnp.float32),
            pltpu.SemaphoreType.DMA,
        ],
    )
    return k(txt_w, word_embeddings, position_embeddings,
             token_type_embeddings, ln_gamma, ln_beta)


def kernel(txt, word_embeddings, position_embeddings, token_type_embeddings,
           ln_gamma, ln_beta):
    txt_w = txt.reshape(B, NW, POS_W).transpose(1, 0, 2).reshape(NW, TOK_W)
    out = _run(txt_w, word_embeddings, position_embeddings,
               token_type_embeddings, ln_gamma, ln_beta)
    return out.reshape(B, L, D)


# speedup vs baseline: 1.5915x; 1.5915x over previous
import jax
import jax.numpy as jnp
from jax import lax
from jax.experimental import pallas as pl
from jax.experimental.pallas import tpu as pltpu
from jax.experimental.pallas import tpu_sc as plsc

D = 768
KV = D // 16
B = 128
L = 512
NC = 2
NS = 16
NW = NC * NS
POS_W = L // NW
TOK_W = B * POS_W
SEQ_CHUNK = 2
CROWS = SEQ_CHUNK * POS_W
NCHUNK = B // SEQ_CHUNK
NBUF = 4
EPS = 1e-12


def _body(txt_ref, we_ref, pe_ref, te_ref, g_ref, b_ref, out_ref,
          idx_v, pe_v, te_v, g_v, b_v,
          buf0, buf1, buf2, buf3,
          gs0, gs1, gs2, gs3, os0, os1, os2, os3):
    c = lax.axis_index("c")
    s = lax.axis_index("s")
    wid = s * NC + c
    bufs = [buf0, buf1, buf2, buf3]
    gsems = [gs0, gs1, gs2, gs3]
    osems = [os0, os1, os2, os3]

    pltpu.sync_copy(txt_ref.at[wid], idx_v)
    pltpu.sync_copy(pe_ref.at[pl.ds(wid * POS_W, POS_W)], pe_v)
    pltpu.sync_copy(te_ref.at[pl.ds(0, 1)], te_v)
    pltpu.sync_copy(g_ref, g_v)
    pltpu.sync_copy(b_ref, b_v)

    def fold(r, carry):
        for k in range(KV):
            sl = pl.ds(k * 16, 16)
            pe_v[r, sl] = pe_v[r, sl] + te_v[0, sl]
        return carry
    lax.fori_loop(0, POS_W, fold, 0)

    def gather(ch, b):
        return pltpu.make_async_copy(
            we_ref.at[idx_v.at[pl.ds(ch * CROWS, CROWS)]], bufs[b], gsems[b])

    def out_copy(ch, j, b):
        seq = ch * SEQ_CHUNK + j
        return pltpu.make_async_copy(
            bufs[b].at[pl.ds(j * POS_W, POS_W)],
            out_ref.at[pl.ds(seq * L + wid * POS_W, POS_W)],
            osems[b])

    inv_d = jnp.float32(1.0 / D)

    def make_row_ln(buf):
        def row_ln(r, carry):
            p = jnp.bitwise_and(r, POS_W - 1)
            acc = jnp.zeros((16,), jnp.float32)
            acc2 = jnp.zeros((16,), jnp.float32)
            xs = []
            for k in range(KV):
                sl = pl.ds(k * 16, 16)
                x = buf[r, sl] + pe_v[p, sl]
                xs.append(x)
                acc = acc + x
                acc2 = acc2 + x * x
            s1 = jnp.sum(acc)
            s2 = jnp.sum(acc2)
            mu = s1 * inv_d
            var = s2 * inv_d - mu * mu + jnp.float32(EPS)
            v = jnp.full((16,), var, jnp.float32)
            i = plsc.bitcast(v, jnp.int32)
            y = plsc.bitcast(jnp.int32(0x5F3759DF) - (i >> 1), jnp.float32)
            half_v = v * jnp.float32(0.5)
            for _ in range(3):
                y = y * (jnp.float32(1.5) - half_v * y * y)
            muv = jnp.full((16,), mu, jnp.float32)
            for k in range(KV):
                sl = pl.ds(k * 16, 16)
                buf[r, sl] = (xs[k] - muv) * y * g_v[sl] + b_v[sl]
            return carry
        return row_ln

    for b in range(NBUF - 1):
        gather(b, b).start()

    def outer(o, carry):
        for b in range(NBUF):
            ch = o * NBUF + b
            gather(ch, b).wait()
            lax.fori_loop(0, CROWS, make_row_ln(bufs[b]), 0)
            for j in range(SEQ_CHUNK):
                out_copy(ch, j, b).start()
            chn = ch + NBUF - 1
            bn = (b + NBUF - 1) % NBUF

            def prefetch():
                for j in range(SEQ_CHUNK):
                    out_copy(chn - NBUF, j, bn).wait()
                gather(chn, bn).start()

            if b == 0:
                @pl.when(o == 0)
                def _():
                    gather(chn, bn).start()

                @pl.when(jnp.logical_and(o > 0, chn < NCHUNK))
                def _():
                    prefetch()
            else:
                @pl.when(chn < NCHUNK)
                def _():
                    prefetch()
        return carry

    lax.fori_loop(0, NCHUNK // NBUF, outer, 0)

    for b in range(NBUF):
        ch = NCHUNK - NBUF + b
        for j in range(SEQ_CHUNK):
            out_copy(ch, j, b).wait()


@jax.jit
def _run(txt_w, word_embeddings, position_embeddings, token_type_embeddings,
         ln_gamma, ln_beta):
    mesh = plsc.VectorSubcoreMesh(core_axis_name="c", subcore_axis_name="s")
    k = pl.kernel(
        _body,
        out_type=jax.ShapeDtypeStruct((B * L, D), jnp.float32),
        mesh=mesh,
        compiler_params=pltpu.CompilerParams(needs_layout_passes=False),
        scratch_types=[
            pltpu.VMEM((TOK_W,), jnp.int32),
            pltpu.VMEM((POS_W, D), jnp.float32),
            pltpu.VMEM((1, D), jnp.float32),
            pltpu.VMEM((D,), jnp.float32),
            pltpu.VMEM((D,), jnp.float32),
        ] + [pltpu.VMEM((CROWS, D), jnp.float32)] * NBUF
          + [pltpu.SemaphoreType.DMA] * (2 * NBUF),
    )
    return k(txt_w, word_embeddings, position_embeddings,
             token_type_embeddings, ln_gamma, ln_beta)


def kernel(txt, word_embeddings, position_embeddings, token_type_embeddings,
           ln_gamma, ln_beta):
    txt_w = txt.reshape(B, NW, POS_W).transpose(1, 0, 2).reshape(NW, TOK_W)
    out = _run(txt_w, word_embeddings, position_embeddings,
               token_type_embeddings, ln_gamma, ln_beta)
    return out.reshape(B, L, D)


# speedup vs baseline: 3.7916x; 2.3824x over previous
import jax
import jax.numpy as jnp
from jax import lax
from jax.experimental import pallas as pl
from jax.experimental.pallas import tpu as pltpu
from jax.experimental.pallas import tpu_sc as plsc

D = 768
B = 128
L = 512
N = B * L
NC = 2
NS = 16
NW = NC * NS
TOK_W = N // NW
CROWS = 32
NCHUNK = TOK_W // CROWS
NBUF = 4
EPS = 1e-12
BLK = 512


def _gather_body(txt_ref, we_ref, out_ref, idx_v,
                 buf0, buf1, buf2, buf3,
                 gs0, gs1, gs2, gs3, os0, os1, os2, os3):
    c = lax.axis_index("c")
    s = lax.axis_index("s")
    wid = s * NC + c
    base = wid * TOK_W
    bufs = [buf0, buf1, buf2, buf3]
    gsems = [gs0, gs1, gs2, gs3]
    osems = [os0, os1, os2, os3]

    pltpu.sync_copy(txt_ref.at[pl.ds(base, TOK_W)], idx_v)

    def gather(ch, b):
        return pltpu.make_async_copy(
            we_ref.at[idx_v.at[pl.ds(ch * CROWS, CROWS)]], bufs[b], gsems[b])

    def out_copy(ch, b):
        return pltpu.make_async_copy(
            bufs[b], out_ref.at[pl.ds(base + ch * CROWS, CROWS)], osems[b])

    for b in range(NBUF - 1):
        gather(b, b).start()

    def outer(o, carry):
        for b in range(NBUF):
            ch = o * NBUF + b
            gather(ch, b).wait()
            out_copy(ch, b).start()
            chn = ch + NBUF - 1
            bn = (b + NBUF - 1) % NBUF

            def prefetch():
                out_copy(chn - NBUF, bn).wait()
                gather(chn, bn).start()

            if b == 0:
                @pl.when(o == 0)
                def _():
                    gather(chn, bn).start()

                @pl.when(jnp.logical_and(o > 0, chn < NCHUNK))
                def _():
                    prefetch()
            else:
                @pl.when(chn < NCHUNK)
                def _():
                    prefetch()
        return carry

    lax.fori_loop(0, NCHUNK // NBUF, outer, 0)

    for b in range(NBUF):
        out_copy(NCHUNK - NBUF + b, b).wait()


def _ln_body(we_ref, pe_ref, te_ref, g_ref, b_ref, o_ref):
    x = we_ref[...] + pe_ref[...] + te_ref[...]
    mu = jnp.mean(x, axis=-1, keepdims=True)
    var = jnp.mean(x * x, axis=-1, keepdims=True) - mu * mu
    y = (x - mu) * lax.rsqrt(var + EPS)
    o_ref[...] = y * g_ref[...] + b_ref[...]


@jax.jit
def _run(txt_flat, word_embeddings, position_embeddings,
         token_type_embeddings, ln_gamma, ln_beta):
    mesh = plsc.VectorSubcoreMesh(core_axis_name="c", subcore_axis_name="s")
    gathered = pl.kernel(
        _gather_body,
        out_type=jax.ShapeDtypeStruct((N, D), jnp.float32),
        mesh=mesh,
        compiler_params=pltpu.CompilerParams(needs_layout_passes=False),
        scratch_types=[pltpu.VMEM((TOK_W,), jnp.int32)]
        + [pltpu.VMEM((CROWS, D), jnp.float32)] * NBUF
        + [pltpu.SemaphoreType.DMA] * (2 * NBUF),
    )(txt_flat, word_embeddings)

    pe_rep = position_embeddings
    out = pl.pallas_call(
        _ln_body,
        out_shape=jax.ShapeDtypeStruct((N, D), jnp.float32),
        grid=(N // BLK,),
        in_specs=[
            pl.BlockSpec((BLK, D), lambda i: (i, 0)),
            pl.BlockSpec((L, D), lambda i: (0, 0)),
            pl.BlockSpec((1, D), lambda i: (0, 0)),
            pl.BlockSpec((1, D), lambda i: (0, 0)),
            pl.BlockSpec((1, D), lambda i: (0, 0)),
        ],
        out_specs=pl.BlockSpec((BLK, D), lambda i: (i, 0)),
        compiler_params=pltpu.CompilerParams(
            dimension_semantics=("arbitrary",)),
    )(gathered, pe_rep, token_type_embeddings[:1],
      ln_gamma.reshape(1, D), ln_beta.reshape(1, D))
    return out


def kernel(txt, word_embeddings, position_embeddings, token_type_embeddings,
           ln_gamma, ln_beta):
    out = _run(txt.reshape(N), word_embeddings, position_embeddings,
               token_type_embeddings, ln_gamma, ln_beta)
    return out.reshape(B, L, D)


# speedup vs baseline: 4.0686x; 1.0731x over previous
import jax
import jax.numpy as jnp
from jax import lax
from jax.experimental import pallas as pl
from jax.experimental.pallas import tpu as pltpu
from jax.experimental.pallas import tpu_sc as plsc

D = 768
B = 128
L = 512
N = B * L
NC = 2
NS = 16
NW = NC * NS
NSLICE = 4
NTOK_S = N // NSLICE
TOK_W = NTOK_S // NW
CROWS = 32
NCHUNK = TOK_W // CROWS
NBUF = 4
EPS = 1e-12
BLK = 512
GRID_S = NTOK_S // BLK


def _gather_body(txt_ref, we_ref, out_ref, idx_v,
                 buf0, buf1, buf2, buf3,
                 gs0, gs1, gs2, gs3, os0, os1, os2, os3):
    c = lax.axis_index("c")
    s = lax.axis_index("s")
    wid = s * NC + c
    base = wid * TOK_W
    bufs = [buf0, buf1, buf2, buf3]
    gsems = [gs0, gs1, gs2, gs3]
    osems = [os0, os1, os2, os3]

    pltpu.sync_copy(txt_ref.at[pl.ds(base, TOK_W)], idx_v)

    def gather(ch, b):
        return pltpu.make_async_copy(
            we_ref.at[idx_v.at[pl.ds(ch * CROWS, CROWS)]], bufs[b], gsems[b])

    def out_copy(ch, b):
        return pltpu.make_async_copy(
            bufs[b], out_ref.at[pl.ds(base + ch * CROWS, CROWS)], osems[b])

    for b in range(NBUF - 1):
        gather(b, b).start()

    def outer(o, carry):
        for b in range(NBUF):
            ch = o * NBUF + b
            gather(ch, b).wait()
            out_copy(ch, b).start()
            chn = ch + NBUF - 1
            bn = (b + NBUF - 1) % NBUF

            def prefetch():
                out_copy(chn - NBUF, bn).wait()
                gather(chn, bn).start()

            if b == 0:
                @pl.when(o == 0)
                def _():
                    gather(chn, bn).start()

                @pl.when(jnp.logical_and(o > 0, chn < NCHUNK))
                def _():
                    prefetch()
            else:
                @pl.when(chn < NCHUNK)
                def _():
                    prefetch()
        return carry

    lax.fori_loop(0, NCHUNK // NBUF, outer, 0)

    for b in range(NBUF):
        out_copy(NCHUNK - NBUF + b, b).wait()


def _sc_gather(txt_slice, word_embeddings):
    mesh = plsc.VectorSubcoreMesh(core_axis_name="c", subcore_axis_name="s")
    return pl.kernel(
        _gather_body,
        out_type=jax.ShapeDtypeStruct((NTOK_S, D), jnp.float32),
        mesh=mesh,
        compiler_params=pltpu.CompilerParams(needs_layout_passes=False),
        scratch_types=[pltpu.VMEM((TOK_W,), jnp.int32)]
        + [pltpu.VMEM((CROWS, D), jnp.float32)] * NBUF
        + [pltpu.SemaphoreType.DMA] * (2 * NBUF),
    )(txt_slice, word_embeddings)


def _ln_first_body(we_ref, pe_ref, te_ref, g_ref, b_ref, o_ref):
    x = we_ref[...] + pe_ref[...] + te_ref[...]
    mu = jnp.mean(x, axis=-1, keepdims=True)
    var = jnp.mean(x * x, axis=-1, keepdims=True) - mu * mu
    y = (x - mu) * lax.rsqrt(var + EPS)
    o_ref[...] = y * g_ref[...] + b_ref[...]


def _ln_chain_body(we_ref, pe_ref, te_ref, g_ref, b_ref, prev_ref, o_ref):
    _ln_first_body(we_ref, pe_ref, te_ref, g_ref, b_ref, o_ref)


def _tc_ln(sl, gathered, pe, te1, g2, b2, prev):
    in_specs = [
        pl.BlockSpec((BLK, D), lambda i: (i, 0)),
        pl.BlockSpec((L, D), lambda i: (0, 0)),
        pl.BlockSpec((1, D), lambda i: (0, 0)),
        pl.BlockSpec((1, D), lambda i: (0, 0)),
        pl.BlockSpec((1, D), lambda i: (0, 0)),
    ]
    args = [gathered, pe, te1, g2, b2]
    body = _ln_first_body
    aliases = {}
    if prev is not None:
        in_specs.append(pl.BlockSpec((8, D), lambda i: (0, 0)))
        args.append(prev)
        body = _ln_chain_body
        aliases = {5: 0}
    base_blk = sl * GRID_S
    return pl.pallas_call(
        body,
        out_shape=jax.ShapeDtypeStruct((N, D), jnp.float32),
        grid=(GRID_S,),
        in_specs=in_specs,
        out_specs=pl.BlockSpec((BLK, D), lambda i, _b=base_blk: (i + _b, 0)),
        input_output_aliases=aliases,
        compiler_params=pltpu.CompilerParams(
            dimension_semantics=("arbitrary",)),
    )(*args)


@jax.jit
def _run(txt_flat, word_embeddings, position_embeddings,
         token_type_embeddings, ln_gamma, ln_beta):
    te1 = token_type_embeddings[:1]
    g2 = ln_gamma.reshape(1, D)
    b2 = ln_beta.reshape(1, D)
    gathered = [
        _sc_gather(lax.slice(txt_flat, (sl * NTOK_S,), ((sl + 1) * NTOK_S,)),
                   word_embeddings)
        for sl in range(NSLICE)
    ]
    out = None
    for sl in range(NSLICE):
        out = _tc_ln(sl, gathered[sl], position_embeddings, te1, g2, b2, out)
    return out


def kernel(txt, word_embeddings, position_embeddings, token_type_embeddings,
           ln_gamma, ln_beta):
    out = _run(txt.reshape(N), word_embeddings, position_embeddings,
               token_type_embeddings, ln_gamma, ln_beta)
    return out.reshape(B, L, D)
